# Initial kernel scaffold; baseline (speedup 1.0000x reference)
import jax
import jax.numpy as jnp
from jax.experimental import pallas as pl


def kernel(frame_indices, point_clouds, mask):
    raise NotImplementedError("write your pallas kernel here")


# speedup vs baseline: 10.5622x; 10.5622x over previous
import functools

import jax
import jax.numpy as jnp
from jax import lax
from jax.experimental import pallas as pl
from jax.experimental.pallas import tpu as pltpu
from jax.experimental.pallas import tpu_sc as plsc

EPS = 1e-06

B = 32
L = 16384
N = 16384
CH = 2048
GROUPS = CH // 16
NCHUNK = L // CH


def _nsqrt(s):
    bits = lax.bitcast_convert_type(s, jnp.int32)
    y = lax.bitcast_convert_type(
        lax.shift_right_logical(bits, 1) + jnp.int32(0x1FBD1DF5), jnp.float32)
    y = 0.5 * (y + s / y)
    y = 0.5 * (y + s / y)
    y = 0.5 * (y + s / y)
    return y


def _frame_body(idx_hbm, pts_hbm, mask_hbm, frames_hbm, mout_hbm,
                tbl_v, idx_v, msk_v, out_v, mout_v):
    nc = 2
    wid = lax.axis_index("s") * nc + lax.axis_index("c")

    pltpu.sync_copy(pts_hbm.at[wid], tbl_v)

    lanes = lax.iota(jnp.int32, 16)

    def do_chunk(c, _):
        c0 = c * CH
        pltpu.sync_copy(idx_hbm.at[wid, pl.ds(c0 * 3, CH * 3)], idx_v)
        pltpu.sync_copy(mask_hbm.at[wid, pl.ds(c0, CH)], msk_v)

        def group(g, carry):
            b3 = g * 48
            l3 = lanes * 3 + b3
            i0 = plsc.load_gather(idx_v, [l3])
            i1 = plsc.load_gather(idx_v, [l3 + 1])
            i2 = plsc.load_gather(idx_v, [l3 + 2])
            i0 = jnp.minimum(jnp.maximum(i0, 0), N - 1) * 3
            i1 = jnp.minimum(jnp.maximum(i1, 0), N - 1) * 3
            i2 = jnp.minimum(jnp.maximum(i2, 0), N - 1) * 3

            p0x = plsc.load_gather(tbl_v, [i0])
            p0y = plsc.load_gather(tbl_v, [i0 + 1])
            p0z = plsc.load_gather(tbl_v, [i0 + 2])
            p1x = plsc.load_gather(tbl_v, [i1])
            p1y = plsc.load_gather(tbl_v, [i1 + 1])
            p1z = plsc.load_gather(tbl_v, [i1 + 2])
            p2x = plsc.load_gather(tbl_v, [i2])
            p2y = plsc.load_gather(tbl_v, [i2 + 1])
            p2z = plsc.load_gather(tbl_v, [i2 + 2])

            dx, dy, dz = p1x - p0x, p1y - p0y, p1z - p0z
            ex, ey, ez = p2x - p0x, p2y - p0y, p2z - p0z

            den = _nsqrt(dx * dx + dy * dy + dz * dz) + EPS
            azx = dx / den
            azy = dy / den
            azz = (dz + EPS) / den

            cxx = azy * ez - azz * ey
            cxy = azz * ex - azx * ez
            cxz = azx * ey - azy * ex
            den = _nsqrt(cxx * cxx + cxy * cxy + cxz * cxz) + EPS
            ayx = cxx / den
            ayy = (cxy + EPS) / den
            ayz = cxz / den

            bxx = ayy * azz - ayz * azy
            bxy = ayz * azx - ayx * azz
            bxz = ayx * azy - ayy * azx
            den = _nsqrt(bxx * bxx + bxy * bxy + bxz * bxz) + EPS
            axx = (bxx + EPS) / den
            axy = bxy / den
            axz = bxz / den

            sb = lanes * 12 + g * 192
            plsc.store_scatter(out_v, [sb], p0x)
            plsc.store_scatter(out_v, [sb + 1], p0y)
            plsc.store_scatter(out_v, [sb + 2], p0z)
            plsc.store_scatter(out_v, [sb + 3], axx)
            plsc.store_scatter(out_v, [sb + 4], axy)
            plsc.store_scatter(out_v, [sb + 5], axz)
            plsc.store_scatter(out_v, [sb + 6], ayx)
            plsc.store_scatter(out_v, [sb + 7], ayy)
            plsc.store_scatter(out_v, [sb + 8], ayz)
            plsc.store_scatter(out_v, [sb + 9], azx)
            plsc.store_scatter(out_v, [sb + 10], azy)
            plsc.store_scatter(out_v, [sb + 11], azz)

            mv = plsc.load_gather(msk_v, [lanes + g * 16])
            mb = lanes * 4 + g * 64
            plsc.store_scatter(mout_v, [mb], mv)
            plsc.store_scatter(mout_v, [mb + 1], mv)
            plsc.store_scatter(mout_v, [mb + 2], mv)
            plsc.store_scatter(mout_v, [mb + 3], mv)
            return carry

        lax.fori_loop(0, GROUPS, group, 0)

        pltpu.sync_copy(out_v, frames_hbm.at[wid, pl.ds(c0 * 12, CH * 12)])
        pltpu.sync_copy(mout_v, mout_hbm.at[wid, pl.ds(c0 * 4, CH * 4)])
        return _

    lax.fori_loop(0, NCHUNK, do_chunk, 0)


_mesh = plsc.VectorSubcoreMesh(core_axis_name="c", subcore_axis_name="s")

_frame_call = functools.partial(
    pl.kernel,
    mesh=_mesh,
    compiler_params=pltpu.CompilerParams(needs_layout_passes=False),
    out_type=[
        jax.ShapeDtypeStruct((B, L * 12), jnp.float32),
        jax.ShapeDtypeStruct((B, L * 4), jnp.float32),
    ],
    scratch_types=[
        pltpu.VMEM((N * 3,), jnp.float32),
        pltpu.VMEM((CH * 3,), jnp.int32),
        pltpu.VMEM((CH,), jnp.float32),
        pltpu.VMEM((CH * 12,), jnp.float32),
        pltpu.VMEM((CH * 4,), jnp.float32),
    ],
)(_frame_body)


def kernel(frame_indices, point_clouds, mask):
    idx_flat = frame_indices.reshape(B, L * 3)
    pts_flat = point_clouds.reshape(B, N * 3)
    mask2 = mask[0]
    frames_flat, mout_flat = _frame_call(idx_flat, pts_flat, mask2)
    return frames_flat.reshape(B, L, 4, 3), mout_flat.reshape(B, L, 4)


# speedup vs baseline: 61.1110x; 5.7858x over previous
import functools

import jax
import jax.numpy as jnp
from jax import lax
from jax.experimental import pallas as pl
from jax.experimental.pallas import tpu as pltpu
from jax.experimental.pallas import tpu_sc as plsc

EPS = 1e-06

B = 32
L = 16384
N = 16384
CH = 2048
LB = CH // 128
GROUPS = CH // 16
NCHUNK = L // CH


def _inv_den(s):
    s = jnp.maximum(s, 1e-30)
    bits = lax.bitcast_convert_type(s, jnp.int32)
    r = lax.bitcast_convert_type(
        jnp.int32(0x5F3759DF) - lax.shift_right_logical(bits, 1), jnp.float32)
    hs = 0.5 * s
    r = r * (1.5 - hs * r * r)
    r = r * (1.5 - hs * r * r)
    r = r * (1.5 - hs * r * r)
    return 1.0 / (s * r + EPS)


def _frame_body(idx_hbm, pts_hbm, mask_hbm, f_hbm, m_hbm,
                tbl_v, idx_v, msk_v, out_v, mout_v):
    nc = 2
    wid = lax.axis_index("s") * nc + lax.axis_index("c")
    bb = wid // 8
    bi = wid % 8

    for c in range(3):
        pltpu.sync_copy(pts_hbm.at[c, bb, :, bi, :], tbl_v.at[c])
    tbl2 = tbl_v.reshape(3, N)
    c0v = jnp.zeros((16,), jnp.int32)
    c1v = jnp.full((16,), 1, jnp.int32)
    c2v = jnp.full((16,), 2, jnp.int32)

    lanes = lax.iota(jnp.int32, 16)

    for t in range(NCHUNK):
        for c in range(3):
            pltpu.sync_copy(idx_hbm.at[c, bb, pl.ds(t * LB, LB), bi, :],
                            idx_v.at[c])
        pltpu.sync_copy(mask_hbm.at[bb, pl.ds(t * LB, LB), bi, :], msk_v)
        idx2 = idx_v.reshape(3, CH)
        msk2 = msk_v.reshape(1, CH)

        @plsc.parallel_loop(0, GROUPS, unroll=2)
        def group(g):
            lbl = lax.shift_right_logical(g, 3)
            fbase = lbl * 128 + (g & 7) * 16
            i0 = idx2[0, pl.ds(fbase, 16)]
            i1 = idx2[1, pl.ds(fbase, 16)]
            i2 = idx2[2, pl.ds(fbase, 16)]
            mv = msk2[0, pl.ds(fbase, 16)]
            i0 = jnp.minimum(jnp.maximum(i0, 0), N - 1)
            i1 = jnp.minimum(jnp.maximum(i1, 0), N - 1)
            i2 = jnp.minimum(jnp.maximum(i2, 0), N - 1)

            p0x = plsc.load_gather(tbl2, [c0v, i0])
            p0y = plsc.load_gather(tbl2, [c1v, i0])
            p0z = plsc.load_gather(tbl2, [c2v, i0])
            p1x = plsc.load_gather(tbl2, [c0v, i1])
            p1y = plsc.load_gather(tbl2, [c1v, i1])
            p1z = plsc.load_gather(tbl2, [c2v, i1])
            p2x = plsc.load_gather(tbl2, [c0v, i2])
            p2y = plsc.load_gather(tbl2, [c1v, i2])
            p2z = plsc.load_gather(tbl2, [c2v, i2])

            dx, dy, dz = p1x - p0x, p1y - p0y, p1z - p0z
            ex, ey, ez = p2x - p0x, p2y - p0y, p2z - p0z

            inv = _inv_den(dx * dx + dy * dy + dz * dz)
            azx = dx * inv
            azy = dy * inv
            azz = (dz + EPS) * inv

            cxx = azy * ez - azz * ey
            cxy = azz * ex - azx * ez
            cxz = azx * ey - azy * ex
            inv = _inv_den(cxx * cxx + cxy * cxy + cxz * cxz)
            ayx = cxx * inv
            ayy = (cxy + EPS) * inv
            ayz = cxz * inv

            bxx = ayy * azz - ayz * azy
            bxy = ayz * azx - ayx * azz
            bxz = ayx * azy - ayy * azx
            inv = _inv_den(bxx * bxx + bxy * bxy + bxz * bxz)
            axx = (bxx + EPS) * inv
            axy = bxy * inv
            axz = bxz * inv

            ob = lbl * 512 + (g & 7) * 16
            out2 = out_v.reshape(3, 4 * CH)
            mout2 = mout_v.reshape(1, 4 * CH)
            out2[0, pl.ds(ob, 16)] = p0x
            out2[0, pl.ds(ob + 128, 16)] = axx
            out2[0, pl.ds(ob + 256, 16)] = ayx
            out2[0, pl.ds(ob + 384, 16)] = azx
            out2[1, pl.ds(ob, 16)] = p0y
            out2[1, pl.ds(ob + 128, 16)] = axy
            out2[1, pl.ds(ob + 256, 16)] = ayy
            out2[1, pl.ds(ob + 384, 16)] = azy
            out2[2, pl.ds(ob, 16)] = p0z
            out2[2, pl.ds(ob + 128, 16)] = axz
            out2[2, pl.ds(ob + 256, 16)] = ayz
            out2[2, pl.ds(ob + 384, 16)] = azz

            mout2[0, pl.ds(ob, 16)] = mv
            mout2[0, pl.ds(ob + 128, 16)] = mv
            mout2[0, pl.ds(ob + 256, 16)] = mv
            mout2[0, pl.ds(ob + 384, 16)] = mv

        pltpu.sync_copy(out_v, f_hbm.at[wid, :, pl.ds(t * 4 * LB, 4 * LB), :])
        pltpu.sync_copy(mout_v, m_hbm.at[wid, pl.ds(t * 4 * LB, 4 * LB), :])


_mesh = plsc.VectorSubcoreMesh(core_axis_name="c", subcore_axis_name="s")

_frame_call = functools.partial(
    pl.kernel,
    mesh=_mesh,
    compiler_params=pltpu.CompilerParams(needs_layout_passes=False),
    out_type=[
        jax.ShapeDtypeStruct((B, 3, L // 128 * 4, 128), jnp.float32),
        jax.ShapeDtypeStruct((B, L // 128 * 4, 128), jnp.float32),
    ],
    scratch_types=[
        pltpu.VMEM((3, 128, 128), jnp.float32),
        pltpu.VMEM((3, LB, 128), jnp.int32),
        pltpu.VMEM((LB, 128), jnp.float32),
        pltpu.VMEM((3, 4 * LB, 128), jnp.float32),
        pltpu.VMEM((4 * LB, 128), jnp.float32),
    ],
)(_frame_body)


def kernel(frame_indices, point_clouds, mask):
    idx5 = frame_indices.transpose(2, 0, 1).reshape(3, 4, 8, 128, 128)
    idx5 = idx5.transpose(0, 1, 3, 2, 4)
    pts5 = point_clouds.transpose(2, 0, 1).reshape(3, 4, 8, 128, 128)
    pts5 = pts5.transpose(0, 1, 3, 2, 4)
    m5 = mask.reshape(4, 8, 128, 128).transpose(0, 2, 1, 3)

    f4, m4 = _frame_call(idx5, pts5, m5)

    frames = f4.reshape(B, 3, 128, 4, 128).transpose(0, 2, 4, 3, 1)
    frames = frames.reshape(B, L, 4, 3)
    mask_out = m4.reshape(B, 128, 4, 128).transpose(0, 1, 3, 2)
    mask_out = mask_out.reshape(B, L, 4)
    return frames, mask_out
